# baseline (speedup 1.0000x reference)
import functools

import jax
import jax.numpy as jnp
from jax import lax
from jax.experimental import pallas as pl
from jax.experimental.pallas import tpu as pltpu, tpu_sc as plsc

VOCAB = 1000000
D = 32
MAX_LEN = 200
BATCH = 16384
SEQ = 200
TOTAL = BATCH * SEQ

NC, NS = 2, 16
NW = NC * NS
G = 128
C = 512
NG = C // G
PER_W = TOTAL // NW
CHUNKS = PER_W // C
GU_PER_W = PER_W // G


def _emb_kernel(tok_hbm, pos_hbm, item_hbm, ptab_hbm, out_hbm,
                idx_t0, idx_t1, idx_p0, idx_p1, rows0, rows1,
                sbuf0, sbuf1, ptab_v, gsem0, gsem1, wsem0, wsem1):
    wid = lax.axis_index("c") * NS + lax.axis_index("s")
    gu_base = wid * GU_PER_W
    ob_base = wid * (PER_W // 4)

    idx_t = (idx_t0, idx_t1)
    idx_p = (idx_p0, idx_p1)
    rows = (rows0, rows1)
    sbuf = (sbuf0, sbuf1)
    gsem = (gsem0, gsem1)
    wsem = (wsem0, wsem1)

    pltpu.sync_copy(ptab_hbm, ptab_v)

    def stage_and_fire(i, b):
        rb = gu_base + i * NG
        pltpu.sync_copy(tok_hbm.at[pl.ds(rb, NG)], idx_t[b])
        pltpu.sync_copy(pos_hbm.at[pl.ds(rb, NG)], idx_p[b])
        for j in range(NG):
            pltpu.async_copy(item_hbm.at[idx_t[b].at[j]],
                             rows[b].at[pl.ds(j * G, G)], gsem[b])

    def wait_gathers(b):
        for j in range(NG):
            pltpu.make_async_copy(item_hbm.at[idx_t[b].at[j]],
                                  rows[b].at[pl.ds(j * G, G)],
                                  gsem[b]).wait()

    def wb_descr(i, b):
        return pltpu.make_async_copy(
            sbuf[b], out_hbm.at[pl.ds(ob_base + i * (C // 4), C // 4)],
            wsem[b])

    def add_pos(b):
        rbuf, pbuf, obuf = rows[b], idx_p[b], sbuf[b]

        def add_body(gg, carry):
            for j in range(NG):
                p16 = pbuf[j, pl.ds(gg * 16, 16)] * D
                for k in range(16):
                    e = j * G + gg * 16 + k
                    r = j * (G // 4) + gg * 4 + k // 4
                    bofs = p16[k]
                    for d0 in (0, 16):
                        pv = ptab_v[pl.ds(bofs + d0, 16)]
                        iv = rbuf[e, pl.ds(d0, 16)]
                        obuf[r, pl.ds((k % 4) * D + d0, 16)] = iv + pv
            return carry

        lax.fori_loop(0, G // 16, add_body, 0)

    def step(i, b):
        wait_gathers(b)
        add_pos(b)
        wb_descr(i, b).start()
        nb = 1 - b

        @pl.when(i + 1 < CHUNKS)
        def _prefetch():
            stage_and_fire(i + 1, nb)

        @pl.when(i >= 1)
        def _drain_prev_wb():
            wb_descr(i - 1, nb).wait()

    stage_and_fire(0, 0)

    def pair_body(ii, carry):
        step(2 * ii, 0)
        step(2 * ii + 1, 1)
        return carry

    lax.fori_loop(0, CHUNKS // 2, pair_body, 0)
    wb_descr(CHUNKS - 1, (CHUNKS - 1) % 2).wait()


@jax.jit
def kernel(tokens, positions, item_table, pos_table):
    tok2d = tokens.reshape(TOTAL // G, G)
    pos2d = positions.reshape(TOTAL // G, G)

    mesh = plsc.VectorSubcoreMesh(core_axis_name="c", subcore_axis_name="s")
    run = functools.partial(
        pl.kernel,
        out_type=jax.ShapeDtypeStruct((TOTAL // 4, 128), jnp.float32),
        mesh=mesh,
        scratch_types=[
            pltpu.VMEM((NG, G), jnp.int32),
            pltpu.VMEM((NG, G), jnp.int32),
            pltpu.VMEM((NG, G), jnp.int32),
            pltpu.VMEM((NG, G), jnp.int32),
            pltpu.VMEM((C, D), jnp.float32),
            pltpu.VMEM((C, D), jnp.float32),
            pltpu.VMEM((C // 4, 128), jnp.float32),
            pltpu.VMEM((C // 4, 128), jnp.float32),
            pltpu.VMEM((MAX_LEN * D,), jnp.float32),
            pltpu.SemaphoreType.DMA,
            pltpu.SemaphoreType.DMA,
            pltpu.SemaphoreType.DMA,
            pltpu.SemaphoreType.DMA,
        ],
        compiler_params=pltpu.CompilerParams(use_tc_tiling_on_sc=False),
    )(_emb_kernel)
    out128 = run(tok2d, pos2d, item_table, pos_table.reshape(MAX_LEN * D))
    return out128.reshape(BATCH, SEQ, D)


# speedup vs baseline: 1.0867x; 1.0867x over previous
import functools

import jax
import jax.numpy as jnp
from jax import lax
from jax.experimental import pallas as pl
from jax.experimental.pallas import tpu as pltpu, tpu_sc as plsc

VOCAB = 1000000
D = 32
MAX_LEN = 200
BATCH = 16384
SEQ = 200
TOTAL = BATCH * SEQ

NC, NS = 2, 16
NW = NC * NS
G = 128
C = 512
NG = C // G
BT = BATCH // G
BGROUPS = BATCH // C
CHUNKS_TOTAL = SEQ * BGROUPS
CHUNKS = CHUNKS_TOTAL // NW
S_STRIDE = D * BATCH
DT_STRIDE = 8 * BATCH


def _emb_kernel(tok_hbm, pos_hbm, item_hbm, ptab_hbm, out_hbm,
                idx_t0, idx_t1, idx_p0, idx_p1, rows0, rows1,
                sbuf0, sbuf1, ptab_v, gsem0, gsem1, wsem0, wsem1):
    wid = lax.axis_index("c") * NS + lax.axis_index("s")
    q_base = wid * CHUNKS

    idx_t = (idx_t0, idx_t1)
    idx_p = (idx_p0, idx_p1)
    rows = (rows0, rows1)
    sbuf = (sbuf0, sbuf1)
    gsem = (gsem0, gsem1)
    wsem = (wsem0, wsem1)

    pltpu.sync_copy(ptab_hbm, ptab_v)

    iota16 = lax.iota(jnp.int32, 16)
    scat0 = ((iota16 // 8) * 4096) + ((iota16 % 8) * 128)
    scat1 = (((iota16 + 16) // 8) * 4096) + ((iota16 % 8) * 128)

    def stage_and_fire(q, b):
        s = q // BGROUPS
        bg = q % BGROUPS
        rb = s * BT + bg * NG
        pltpu.sync_copy(tok_hbm.at[pl.ds(rb, NG)], idx_t[b])
        pltpu.sync_copy(pos_hbm.at[pl.ds(rb, NG)], idx_p[b])
        for j in range(NG):
            pltpu.async_copy(item_hbm.at[idx_t[b].at[j]],
                             rows[b].at[pl.ds(j * G, G)], gsem[b])

    def wait_gathers(b):
        for j in range(NG):
            pltpu.make_async_copy(item_hbm.at[idx_t[b].at[j]],
                                  rows[b].at[pl.ds(j * G, G)],
                                  gsem[b]).wait()

    def wb_descrs(q, b):
        s = q // BGROUPS
        bg = q % BGROUPS
        base = s * S_STRIDE + bg * (NG * 1024)
        return [
            pltpu.make_async_copy(
                sbuf[b].at[pl.ds(dt * (NG * 1024), NG * 1024)],
                out_hbm.at[pl.ds(base + dt * DT_STRIDE, NG * 1024)],
                wsem[b])
            for dt in range(4)
        ]

    def add_transpose(b):
        rbuf, pbuf, obuf = rows[b], idx_p[b], sbuf[b]

        def body(g, carry):
            p16 = pbuf[g // 8, pl.ds((g % 8) * 16, 16)] * D
            bt_local = g // 8
            bi0 = (g % 8) * 16
            sb_base = bt_local * 1024 + bi0
            for k in range(16):
                e = g * 16 + k
                pofs = p16[k]
                v0 = rbuf[e, pl.ds(0, 16)] + ptab_v[pl.ds(pofs, 16)]
                plsc.store_scatter(obuf, [scat0 + (sb_base + k)], v0)
                v1 = rbuf[e, pl.ds(16, 16)] + ptab_v[pl.ds(pofs + 16, 16)]
                plsc.store_scatter(obuf, [scat1 + (sb_base + k)], v1)
            return carry

        lax.fori_loop(0, C // 16, body, 0)

    def step(i, b):
        q = q_base + i
        wait_gathers(b)
        add_transpose(b)
        for cp in wb_descrs(q, b):
            cp.start()
        nb = 1 - b

        @pl.when(i + 1 < CHUNKS)
        def _prefetch():
            stage_and_fire(q + 1, nb)

        @pl.when(i >= 1)
        def _drain_prev_wb():
            for cp in wb_descrs(q - 1, nb):
                cp.wait()

    stage_and_fire(q_base, 0)

    def pair_body(ii, carry):
        step(2 * ii, 0)
        step(2 * ii + 1, 1)
        return carry

    lax.fori_loop(0, CHUNKS // 2, pair_body, 0)
    for cp in wb_descrs(q_base + CHUNKS - 1, (CHUNKS - 1) % 2):
        cp.wait()


@jax.jit
def kernel(tokens, positions, item_table, pos_table):
    tok_t = tokens.T.reshape(TOTAL // G, G)
    pos_t = positions.T.reshape(TOTAL // G, G)

    mesh = plsc.VectorSubcoreMesh(core_axis_name="c", subcore_axis_name="s")
    run = functools.partial(
        pl.kernel,
        out_type=jax.ShapeDtypeStruct((TOTAL * D,), jnp.float32),
        mesh=mesh,
        scratch_types=[
            pltpu.VMEM((NG, G), jnp.int32),
            pltpu.VMEM((NG, G), jnp.int32),
            pltpu.VMEM((NG, G), jnp.int32),
            pltpu.VMEM((NG, G), jnp.int32),
            pltpu.VMEM((C, D), jnp.float32),
            pltpu.VMEM((C, D), jnp.float32),
            pltpu.VMEM((C * D,), jnp.float32),
            pltpu.VMEM((C * D,), jnp.float32),
            pltpu.VMEM((MAX_LEN * D,), jnp.float32),
            pltpu.SemaphoreType.DMA,
            pltpu.SemaphoreType.DMA,
            pltpu.SemaphoreType.DMA,
            pltpu.SemaphoreType.DMA,
        ],
        compiler_params=pltpu.CompilerParams(use_tc_tiling_on_sc=False,
                                             needs_layout_passes=False),
    )(_emb_kernel)
    out1d = run(tok_t, pos_t, item_table, pos_table.reshape(MAX_LEN * D))
    out5 = out1d.reshape(SEQ, D // 8, BATCH // 128, 8, 128)
    return out5.transpose(2, 4, 0, 1, 3).reshape(BATCH, SEQ, D)
